# baseline (speedup 1.0000x reference)
import functools

import jax
import jax.numpy as jnp
from jax import lax
from jax.experimental import pallas as pl
from jax.experimental.pallas import tpu as pltpu
from jax.experimental.pallas import tpu_sc as plsc

_N = 10000
_E = 320000
_D = 128
_T = 4
_DT = 32
_AVG_D_LOG = 3.4965075614664802
_EPS = 1e-5


_C = 2000
_U = 5
_NPAIR = _E // (2 * _C)
_GPC = _C // 16
_FNEG = -3.0e38
_FPOS = 3.0e38


def _retry_scatter(acc, idx, val, is_max):
    def cond(act):
        return plsc.all_reduce_population_count(act)[0] > 0

    def body(act):
        old = plsc.load_gather(acc, [idx])
        new = jnp.maximum(old, val) if is_max else jnp.minimum(old, val)
        plsc.store_scatter(acc, [idx], new, mask=act)
        chk = plsc.load_gather(acc, [idx])
        lost = (chk < val) if is_max else (chk > val)
        return act & lost

    lax.while_loop(cond, body, jnp.full((16,), True, jnp.bool_))


def _edge_phase(aT, src, dst):
    mesh = plsc.VectorSubcoreMesh(core_axis_name="c", subcore_axis_name="s")
    fvec = lambda v: jnp.full((16,), v, jnp.float32)

    @functools.partial(
        pl.kernel,
        out_type=(
            jax.ShapeDtypeStruct((4 * _D, _N), jnp.float32),
            jax.ShapeDtypeStruct((_N,), jnp.float32),
        ),
        mesh=mesh,
        scratch_types=(
            [pltpu.VMEM((_N,), jnp.float32) for _ in range(11)]
            + [pltpu.VMEM((_C,), jnp.int32) for _ in range(4)]
            + [pltpu.SemaphoreType.DMA for _ in range(4)]
        ),
        compiler_params=pltpu.CompilerParams(needs_layout_passes=False),
    )
    def body(aT_hbm, src_hbm, dst_hbm, agg_out, deg_out,
             sum0, sum1, sq0, sq1, mx0, mx1, mn0, mn1, arow0, arow1, degacc,
             srcA, dstA, srcB, dstB, semAs, semAd, semBs, semBd):
        wid = lax.axis_index("s") * 2 + lax.axis_index("c")

        def process(sbuf, dbuf, deg_on):
            def g_body(i, carry):
                base = i * (16 * _U)
                gs = []
                for u in range(_U):
                    sl = pl.ds(base + u * 16, 16)
                    s16 = sbuf[sl]
                    d16 = dbuf[sl]
                    v0 = plsc.load_gather(arow0, [s16])
                    v1 = plsc.load_gather(arow1, [s16])
                    plsc.addupdate_scatter(sum0, [d16], v0)
                    plsc.addupdate_scatter(sq0, [d16], v0 * v0)
                    plsc.addupdate_scatter(sum1, [d16], v1)
                    plsc.addupdate_scatter(sq1, [d16], v1 * v1)
                    o = plsc.load_gather(mx0, [d16])
                    plsc.store_scatter(mx0, [d16], v0, mask=v0 > o)
                    o = plsc.load_gather(mn0, [d16])
                    plsc.store_scatter(mn0, [d16], v0, mask=v0 < o)
                    o = plsc.load_gather(mx1, [d16])
                    plsc.store_scatter(mx1, [d16], v1, mask=v1 > o)
                    o = plsc.load_gather(mn1, [d16])
                    plsc.store_scatter(mn1, [d16], v1, mask=v1 < o)
                    _, lastm = plsc.scan_count(d16)
                    ndist = plsc.all_reduce_population_count(lastm)[0]
                    gs.append((d16, v0, v1, ndist))

                anydup = gs[0][3] < 16
                for u in range(1, _U):
                    anydup = anydup | (gs[u][3] < 16)

                @pl.when(anydup)
                def _():
                    for d16, v0, v1, _n in gs:
                        _retry_scatter(mx0, d16, v0, True)
                        _retry_scatter(mn0, d16, v0, False)
                        _retry_scatter(mx1, d16, v1, True)
                        _retry_scatter(mn1, d16, v1, False)

                return carry

            lax.fori_loop(0, _GPC // _U, g_body, 0)

            if deg_on:
                @pl.when(wid == 0)
                def _():
                    def deg_body(i, carry):
                        base = i * (16 * _U)
                        for u in range(_U):
                            d16 = dbuf[pl.ds(base + u * 16, 16)]
                            plsc.addupdate_scatter(degacc, [d16], fvec(1.0))
                        return carry

                    lax.fori_loop(0, _GPC // _U, deg_body, 0)

        for p in range(2):
            f0 = p * 64 + 2 * wid
            pltpu.sync_copy(aT_hbm.at[f0], arow0)
            pltpu.sync_copy(aT_hbm.at[f0 + 1], arow1)

            def init_body(i, carry):
                sl = pl.ds(i * 16, 16)
                z = fvec(0.0)
                sum0[sl] = z
                sum1[sl] = z
                sq0[sl] = z
                sq1[sl] = z
                mx0[sl] = fvec(_FNEG)
                mx1[sl] = fvec(_FNEG)
                mn0[sl] = fvec(_FPOS)
                mn1[sl] = fvec(_FPOS)
                if p == 0:
                    degacc[sl] = z
                return carry

            lax.fori_loop(0, _N // 16, init_body, 0)

            def dma(ch, buf_s, buf_d, sem_s, sem_d):
                esl = pl.ds(ch * _C, _C)
                a = pltpu.make_async_copy(src_hbm.at[esl], buf_s, sem_s)
                b = pltpu.make_async_copy(dst_hbm.at[esl], buf_d, sem_d)
                return a, b

            def start(ch, buf_s, buf_d, sem_s, sem_d):
                a, b = dma(ch, buf_s, buf_d, sem_s, sem_d)
                a.start()
                b.start()

            def wait(ch, buf_s, buf_d, sem_s, sem_d):
                a, b = dma(ch, buf_s, buf_d, sem_s, sem_d)
                a.wait()
                b.wait()

            start(0, srcA, dstA, semAs, semAd)

            def pair_body(i, carry):
                start(2 * i + 1, srcB, dstB, semBs, semBd)
                wait(2 * i, srcA, dstA, semAs, semAd)
                process(srcA, dstA, p == 0)

                @pl.when(i < _NPAIR - 1)
                def _():
                    start(2 * i + 2, srcA, dstA, semAs, semAd)

                wait(2 * i + 1, srcB, dstB, semBs, semBd)
                process(srcB, dstB, p == 0)
                return carry

            lax.fori_loop(0, _NPAIR, pair_body, 0)

            pltpu.sync_copy(sum0, agg_out.at[f0])
            pltpu.sync_copy(sum1, agg_out.at[f0 + 1])
            pltpu.sync_copy(sq0, agg_out.at[_D + f0])
            pltpu.sync_copy(sq1, agg_out.at[_D + f0 + 1])
            pltpu.sync_copy(mx0, agg_out.at[2 * _D + f0])
            pltpu.sync_copy(mx1, agg_out.at[2 * _D + f0 + 1])
            pltpu.sync_copy(mn0, agg_out.at[3 * _D + f0])
            pltpu.sync_copy(mn1, agg_out.at[3 * _D + f0 + 1])
            if p == 0:
                @pl.when(wid == 0)
                def _():
                    pltpu.sync_copy(degacc, deg_out)

    return body(aT, src, dst)


def _phase1_body(h_ref, wa_ref, ba_ref, wb_ref, a_ref, b_ref):
    h = h_ref[...]
    a_ref[...] = jnp.dot(h, wa_ref[...], preferred_element_type=jnp.float32) + ba_ref[...]
    b_ref[...] = jnp.dot(h, wb_ref[...], preferred_element_type=jnp.float32)


def _phase1(h, wa, ba, wb):
    return pl.pallas_call(
        _phase1_body,
        out_shape=(
            jax.ShapeDtypeStruct((_N, _D), jnp.float32),
            jax.ShapeDtypeStruct((_N, _D), jnp.float32),
        ),
    )(h, wa, ba, wb)


_NB = 5
_R = _N // _NB


def _tower_body(h_ref, b_ref, ssum_ref, ssq_ref, smax_ref, smin_ref,
                deg_ref, snorm_ref, w0_ref, w1_ref, w2_ref, w3_ref,
                bpost_ref, ppre_ref, sums_ref):
    nb = pl.program_id(0)
    deg = deg_ref[...]
    deg_c = jnp.maximum(deg, 1.0)
    has = deg > 0.0
    log_deg = jnp.log(deg_c + 1.0)
    amp_s = log_deg * (1.0 / _AVG_D_LOG)
    att_s = _AVG_D_LOG / log_deg
    snorm = snorm_ref[...]

    @pl.when(nb == 0)
    def _():
        sums_ref[...] = jnp.zeros_like(sums_ref)

    posts = []
    for t in range(_T):
        sl = slice(t * _DT, (t + 1) * _DT)
        bt = b_ref[:, sl]
        s1 = ssum_ref[:, sl]
        s2 = ssq_ref[:, sl]
        mean = (s1 + deg * bt) / deg_c
        sq = (s2 + 2.0 * bt * s1 + deg * bt * bt) / deg_c
        std = jnp.sqrt(jax.nn.relu(sq - mean * mean) + _EPS)
        mx = jnp.where(has, smax_ref[:, sl] + bt, 0.0)
        mn = jnp.where(has, smin_ref[:, sl] + bt, 0.0)
        agg = jnp.concatenate([mean, mx, mn, std], axis=1)
        post = jnp.dot(h_ref[:, sl], w0_ref[t], preferred_element_type=jnp.float32)
        post += jnp.dot(agg, w1_ref[t], preferred_element_type=jnp.float32)
        post += jnp.dot(agg * amp_s, w2_ref[t], preferred_element_type=jnp.float32)
        post += jnp.dot(agg * att_s, w3_ref[t], preferred_element_type=jnp.float32)
        posts.append((post + bpost_ref[:, sl]) * snorm)
    ppre = jnp.concatenate(posts, axis=1)
    ppre_ref[...] = ppre
    sums_ref[0:1, :] += jnp.sum(ppre, axis=0, keepdims=True)
    sums_ref[1:2, :] += jnp.sum(ppre * ppre, axis=0, keepdims=True)


def _phase2(h, b, ssum, ssq, smax, smin, deg, snorm, w0, w1, w2, w3, bpost):
    col = pl.BlockSpec((_R, _D), lambda i: (i, 0))
    vec = pl.BlockSpec((_R, 1), lambda i: (i, 0))
    fixed3 = pl.BlockSpec((_T, _DT, _DT), lambda i: (0, 0, 0))
    fixedw = pl.BlockSpec((_T, _D, _DT), lambda i: (0, 0, 0))
    return pl.pallas_call(
        _tower_body,
        grid=(_NB,),
        in_specs=[
            col, col, col, col, col, col, vec, vec,
            fixed3, fixedw, fixedw, fixedw,
            pl.BlockSpec((1, _D), lambda i: (0, 0)),
        ],
        out_specs=(
            col,
            pl.BlockSpec((8, _D), lambda i: (0, 0)),
        ),
        out_shape=(
            jax.ShapeDtypeStruct((_N, _D), jnp.float32),
            jax.ShapeDtypeStruct((8, _D), jnp.float32),
        ),
    )(h, b, ssum, ssq, smax, smin, deg, snorm, w0, w1, w2, w3, bpost)


def _phase3_body(h_ref, ppre_ref, sums_ref, gamma_ref, beta_ref, wmix_ref,
                 bmix_ref, out_ref):
    inv_n = 1.0 / _N
    mu = sums_ref[0:1, :] * inv_n
    var = sums_ref[1:2, :] * inv_n - mu * mu
    x = (ppre_ref[...] - mu) * lax.rsqrt(var + _EPS) * gamma_ref[...] \
        + beta_ref[...]
    h_out = jnp.dot(x, wmix_ref[...],
                    preferred_element_type=jnp.float32) + bmix_ref[...]
    h_out = jnp.where(h_out >= 0.0, h_out, 0.01 * h_out)
    out_ref[...] = h_ref[...] + h_out


def _phase3(h, ppre, sums, gamma, beta, wmix, bmix):
    return pl.pallas_call(
        _phase3_body,
        grid=(_NB,),
        in_specs=[
            pl.BlockSpec((_R, _D), lambda i: (i, 0)),
            pl.BlockSpec((_R, _D), lambda i: (i, 0)),
            pl.BlockSpec((8, _D), lambda i: (0, 0)),
            pl.BlockSpec((1, _D), lambda i: (0, 0)),
            pl.BlockSpec((1, _D), lambda i: (0, 0)),
            pl.BlockSpec((_D, _D), lambda i: (0, 0)),
            pl.BlockSpec((_D,), lambda i: (0,)),
        ],
        out_specs=pl.BlockSpec((_R, _D), lambda i: (i, 0)),
        out_shape=jax.ShapeDtypeStruct((_N, _D), jnp.float32),
    )(h, ppre, sums, gamma, beta, wmix, bmix)


def kernel(h, e, snorm_n, W_pre, b_pre, W_post, b_post, gamma, beta, W_mix,
           b_mix, edge_index):
    del e
    wa = jnp.zeros((_D, _D), jnp.float32)
    wb = jnp.zeros((_D, _D), jnp.float32)
    for t in range(_T):
        sl = slice(t * _DT, (t + 1) * _DT)
        wa = wa.at[sl, sl].set(W_pre[t, :_DT])
        wb = wb.at[sl, sl].set(W_pre[t, _DT:])
    ba = b_pre.reshape(_D)

    a, b = _phase1(h, wa, ba, wb)

    src = edge_index[0]
    dst = edge_index[1]
    aggT, deg = _edge_phase(a.T, src, dst)
    ssum = aggT[:_D].T
    ssq = aggT[_D:2 * _D].T
    smax = aggT[2 * _D:3 * _D].T
    smin = aggT[3 * _D:].T

    w0 = W_post[:, :_DT]
    w1 = W_post[:, _DT:_DT + _D]
    w2 = W_post[:, _DT + _D:_DT + 2 * _D]
    w3 = W_post[:, _DT + 2 * _D:]
    ppre, sums = _phase2(h, b, ssum, ssq, smax, smin, deg[:, None], snorm_n,
                         w0, w1, w2, w3, b_post.reshape(1, _D))
    return _phase3(h, ppre, sums, gamma.reshape(1, _D), beta.reshape(1, _D),
                   W_mix, b_mix)


# speedup vs baseline: 1.4778x; 1.4778x over previous
import functools

import jax
import jax.numpy as jnp
from jax import lax
from jax.experimental import pallas as pl
from jax.experimental.pallas import tpu as pltpu
from jax.experimental.pallas import tpu_sc as plsc

_N = 10000
_E = 320000
_D = 128
_T = 4
_DT = 32
_AVG_D_LOG = 3.4965075614664802
_EPS = 1e-5


_C = 2000
_U = 5
_NPAIR = _E // (2 * _C)
_GPC = _C // 16
_FNEG = -3.0e38
_FPOS = 3.0e38


def _retry_scatter(acc, idx, val, is_max):
    def cond(act):
        return plsc.all_reduce_population_count(act)[0] > 0

    def body(act):
        old = plsc.load_gather(acc, [idx])
        new = jnp.maximum(old, val) if is_max else jnp.minimum(old, val)
        plsc.store_scatter(acc, [idx], new, mask=act)
        chk = plsc.load_gather(acc, [idx])
        lost = (chk < val) if is_max else (chk > val)
        return act & lost

    lax.while_loop(cond, body, jnp.full((16,), True, jnp.bool_))


def _edge_phase(aT, src, dst):
    mesh = plsc.VectorSubcoreMesh(core_axis_name="c", subcore_axis_name="s")
    fvec = lambda v: jnp.full((16,), v, jnp.float32)

    @functools.partial(
        pl.kernel,
        out_type=(
            jax.ShapeDtypeStruct((4 * _D, _N), jnp.float32),
            jax.ShapeDtypeStruct((_N,), jnp.float32),
        ),
        mesh=mesh,
        scratch_types=(
            [pltpu.VMEM((_N,), jnp.float32) for _ in range(11)]
            + [pltpu.VMEM((_C,), jnp.int32) for _ in range(4)]
            + [pltpu.SemaphoreType.DMA for _ in range(4)]
        ),
        compiler_params=pltpu.CompilerParams(needs_layout_passes=False),
    )
    def body(aT_hbm, src_hbm, dst_hbm, agg_out, deg_out,
             sum0, sum1, sq0, sq1, mx0, mx1, mn0, mn1, arow0, arow1, degacc,
             srcA, dstA, srcB, dstB, semAs, semAd, semBs, semBd):
        wid = lax.axis_index("s") * 2 + lax.axis_index("c")

        def process(sbuf, dbuf, deg_on):
            def g_body(i, carry):
                base = i * (16 * _U)
                gs = []
                for u in range(_U):
                    sl = pl.ds(base + u * 16, 16)
                    s16 = sbuf[sl]
                    d16 = dbuf[sl]
                    _, lastm = plsc.scan_count(d16)
                    v0 = plsc.load_gather(arow0, [s16])
                    v1 = plsc.load_gather(arow1, [s16])
                    omx0 = plsc.load_gather(mx0, [d16])
                    omn0 = plsc.load_gather(mn0, [d16])
                    omx1 = plsc.load_gather(mx1, [d16])
                    omn1 = plsc.load_gather(mn1, [d16])
                    plsc.addupdate_scatter(sum0, [d16], v0)
                    plsc.addupdate_scatter(sq0, [d16], v0 * v0)
                    plsc.addupdate_scatter(sum1, [d16], v1)
                    plsc.addupdate_scatter(sq1, [d16], v1 * v1)
                    plsc.store_scatter(mx0, [d16], jnp.maximum(omx0, v0))
                    plsc.store_scatter(mn0, [d16], jnp.minimum(omn0, v0))
                    plsc.store_scatter(mx1, [d16], jnp.maximum(omx1, v1))
                    plsc.store_scatter(mn1, [d16], jnp.minimum(omn1, v1))
                    ndist = plsc.all_reduce_population_count(lastm)[0]
                    gs.append((d16, v0, v1, ndist))

                anydup = gs[0][3] < 16
                for u in range(1, _U):
                    anydup = anydup | (gs[u][3] < 16)

                @pl.when(anydup)
                def _():
                    for d16, v0, v1, _n in gs:
                        _retry_scatter(mx0, d16, v0, True)
                        _retry_scatter(mn0, d16, v0, False)
                        _retry_scatter(mx1, d16, v1, True)
                        _retry_scatter(mn1, d16, v1, False)

                return carry

            lax.fori_loop(0, _GPC // _U, g_body, 0)

            if deg_on:
                @pl.when(wid == 0)
                def _():
                    def deg_body(i, carry):
                        base = i * (16 * _U)
                        for u in range(_U):
                            d16 = dbuf[pl.ds(base + u * 16, 16)]
                            plsc.addupdate_scatter(degacc, [d16], fvec(1.0))
                        return carry

                    lax.fori_loop(0, _GPC // _U, deg_body, 0)

        for p in range(2):
            f0 = p * 64 + 2 * wid
            pltpu.sync_copy(aT_hbm.at[f0], arow0)
            pltpu.sync_copy(aT_hbm.at[f0 + 1], arow1)

            def init_body(i, carry):
                sl = pl.ds(i * 16, 16)
                z = fvec(0.0)
                sum0[sl] = z
                sum1[sl] = z
                sq0[sl] = z
                sq1[sl] = z
                mx0[sl] = fvec(_FNEG)
                mx1[sl] = fvec(_FNEG)
                mn0[sl] = fvec(_FPOS)
                mn1[sl] = fvec(_FPOS)
                if p == 0:
                    degacc[sl] = z
                return carry

            lax.fori_loop(0, _N // 16, init_body, 0)

            def dma(ch, buf_s, buf_d, sem_s, sem_d):
                esl = pl.ds(ch * _C, _C)
                a = pltpu.make_async_copy(src_hbm.at[esl], buf_s, sem_s)
                b = pltpu.make_async_copy(dst_hbm.at[esl], buf_d, sem_d)
                return a, b

            def start(ch, buf_s, buf_d, sem_s, sem_d):
                a, b = dma(ch, buf_s, buf_d, sem_s, sem_d)
                a.start()
                b.start()

            def wait(ch, buf_s, buf_d, sem_s, sem_d):
                a, b = dma(ch, buf_s, buf_d, sem_s, sem_d)
                a.wait()
                b.wait()

            start(0, srcA, dstA, semAs, semAd)

            def pair_body(i, carry):
                start(2 * i + 1, srcB, dstB, semBs, semBd)
                wait(2 * i, srcA, dstA, semAs, semAd)
                process(srcA, dstA, p == 0)

                @pl.when(i < _NPAIR - 1)
                def _():
                    start(2 * i + 2, srcA, dstA, semAs, semAd)

                wait(2 * i + 1, srcB, dstB, semBs, semBd)
                process(srcB, dstB, p == 0)
                return carry

            lax.fori_loop(0, _NPAIR, pair_body, 0)

            pltpu.sync_copy(sum0, agg_out.at[f0])
            pltpu.sync_copy(sum1, agg_out.at[f0 + 1])
            pltpu.sync_copy(sq0, agg_out.at[_D + f0])
            pltpu.sync_copy(sq1, agg_out.at[_D + f0 + 1])
            pltpu.sync_copy(mx0, agg_out.at[2 * _D + f0])
            pltpu.sync_copy(mx1, agg_out.at[2 * _D + f0 + 1])
            pltpu.sync_copy(mn0, agg_out.at[3 * _D + f0])
            pltpu.sync_copy(mn1, agg_out.at[3 * _D + f0 + 1])
            if p == 0:
                @pl.when(wid == 0)
                def _():
                    pltpu.sync_copy(degacc, deg_out)

    return body(aT, src, dst)


def _phase1_body(h_ref, wa_ref, ba_ref, wb_ref, a_ref, b_ref):
    h = h_ref[...]
    a_ref[...] = jnp.dot(h, wa_ref[...], preferred_element_type=jnp.float32) + ba_ref[...]
    b_ref[...] = jnp.dot(h, wb_ref[...], preferred_element_type=jnp.float32)


def _phase1(h, wa, ba, wb):
    return pl.pallas_call(
        _phase1_body,
        out_shape=(
            jax.ShapeDtypeStruct((_N, _D), jnp.float32),
            jax.ShapeDtypeStruct((_N, _D), jnp.float32),
        ),
    )(h, wa, ba, wb)


_NB = 5
_R = _N // _NB


def _tower_body(h_ref, b_ref, ssum_ref, ssq_ref, smax_ref, smin_ref,
                deg_ref, snorm_ref, w0_ref, w1_ref, w2_ref, w3_ref,
                bpost_ref, ppre_ref, sums_ref):
    nb = pl.program_id(0)
    deg = deg_ref[...]
    deg_c = jnp.maximum(deg, 1.0)
    has = deg > 0.0
    log_deg = jnp.log(deg_c + 1.0)
    amp_s = log_deg * (1.0 / _AVG_D_LOG)
    att_s = _AVG_D_LOG / log_deg
    snorm = snorm_ref[...]

    @pl.when(nb == 0)
    def _():
        sums_ref[...] = jnp.zeros_like(sums_ref)

    posts = []
    for t in range(_T):
        sl = slice(t * _DT, (t + 1) * _DT)
        bt = b_ref[:, sl]
        s1 = ssum_ref[:, sl]
        s2 = ssq_ref[:, sl]
        mean = (s1 + deg * bt) / deg_c
        sq = (s2 + 2.0 * bt * s1 + deg * bt * bt) / deg_c
        std = jnp.sqrt(jax.nn.relu(sq - mean * mean) + _EPS)
        mx = jnp.where(has, smax_ref[:, sl] + bt, 0.0)
        mn = jnp.where(has, smin_ref[:, sl] + bt, 0.0)
        agg = jnp.concatenate([mean, mx, mn, std], axis=1)
        post = jnp.dot(h_ref[:, sl], w0_ref[t], preferred_element_type=jnp.float32)
        post += jnp.dot(agg, w1_ref[t], preferred_element_type=jnp.float32)
        post += jnp.dot(agg * amp_s, w2_ref[t], preferred_element_type=jnp.float32)
        post += jnp.dot(agg * att_s, w3_ref[t], preferred_element_type=jnp.float32)
        posts.append((post + bpost_ref[:, sl]) * snorm)
    ppre = jnp.concatenate(posts, axis=1)
    ppre_ref[...] = ppre
    sums_ref[0:1, :] += jnp.sum(ppre, axis=0, keepdims=True)
    sums_ref[1:2, :] += jnp.sum(ppre * ppre, axis=0, keepdims=True)


def _phase2(h, b, ssum, ssq, smax, smin, deg, snorm, w0, w1, w2, w3, bpost):
    col = pl.BlockSpec((_R, _D), lambda i: (i, 0))
    vec = pl.BlockSpec((_R, 1), lambda i: (i, 0))
    fixed3 = pl.BlockSpec((_T, _DT, _DT), lambda i: (0, 0, 0))
    fixedw = pl.BlockSpec((_T, _D, _DT), lambda i: (0, 0, 0))
    return pl.pallas_call(
        _tower_body,
        grid=(_NB,),
        in_specs=[
            col, col, col, col, col, col, vec, vec,
            fixed3, fixedw, fixedw, fixedw,
            pl.BlockSpec((1, _D), lambda i: (0, 0)),
        ],
        out_specs=(
            col,
            pl.BlockSpec((8, _D), lambda i: (0, 0)),
        ),
        out_shape=(
            jax.ShapeDtypeStruct((_N, _D), jnp.float32),
            jax.ShapeDtypeStruct((8, _D), jnp.float32),
        ),
    )(h, b, ssum, ssq, smax, smin, deg, snorm, w0, w1, w2, w3, bpost)


def _phase3_body(h_ref, ppre_ref, sums_ref, gamma_ref, beta_ref, wmix_ref,
                 bmix_ref, out_ref):
    inv_n = 1.0 / _N
    mu = sums_ref[0:1, :] * inv_n
    var = sums_ref[1:2, :] * inv_n - mu * mu
    x = (ppre_ref[...] - mu) * lax.rsqrt(var + _EPS) * gamma_ref[...] \
        + beta_ref[...]
    h_out = jnp.dot(x, wmix_ref[...],
                    preferred_element_type=jnp.float32) + bmix_ref[...]
    h_out = jnp.where(h_out >= 0.0, h_out, 0.01 * h_out)
    out_ref[...] = h_ref[...] + h_out


def _phase3(h, ppre, sums, gamma, beta, wmix, bmix):
    return pl.pallas_call(
        _phase3_body,
        grid=(_NB,),
        in_specs=[
            pl.BlockSpec((_R, _D), lambda i: (i, 0)),
            pl.BlockSpec((_R, _D), lambda i: (i, 0)),
            pl.BlockSpec((8, _D), lambda i: (0, 0)),
            pl.BlockSpec((1, _D), lambda i: (0, 0)),
            pl.BlockSpec((1, _D), lambda i: (0, 0)),
            pl.BlockSpec((_D, _D), lambda i: (0, 0)),
            pl.BlockSpec((_D,), lambda i: (0,)),
        ],
        out_specs=pl.BlockSpec((_R, _D), lambda i: (i, 0)),
        out_shape=jax.ShapeDtypeStruct((_N, _D), jnp.float32),
    )(h, ppre, sums, gamma, beta, wmix, bmix)


def kernel(h, e, snorm_n, W_pre, b_pre, W_post, b_post, gamma, beta, W_mix,
           b_mix, edge_index):
    del e
    wa = jnp.zeros((_D, _D), jnp.float32)
    wb = jnp.zeros((_D, _D), jnp.float32)
    for t in range(_T):
        sl = slice(t * _DT, (t + 1) * _DT)
        wa = wa.at[sl, sl].set(W_pre[t, :_DT])
        wb = wb.at[sl, sl].set(W_pre[t, _DT:])
    ba = b_pre.reshape(_D)

    a, b = _phase1(h, wa, ba, wb)

    src = edge_index[0]
    dst = edge_index[1]
    aggT, deg = _edge_phase(a.T, src, dst)
    ssum = aggT[:_D].T
    ssq = aggT[_D:2 * _D].T
    smax = aggT[2 * _D:3 * _D].T
    smin = aggT[3 * _D:].T

    w0 = W_post[:, :_DT]
    w1 = W_post[:, _DT:_DT + _D]
    w2 = W_post[:, _DT + _D:_DT + 2 * _D]
    w3 = W_post[:, _DT + 2 * _D:]
    ppre, sums = _phase2(h, b, ssum, ssq, smax, smin, deg[:, None], snorm_n,
                         w0, w1, w2, w3, b_post.reshape(1, _D))
    return _phase3(h, ppre, sums, gamma.reshape(1, _D), beta.reshape(1, _D),
                   W_mix, b_mix)


# speedup vs baseline: 1.5533x; 1.0511x over previous
import functools

import jax
import jax.numpy as jnp
from jax import lax
from jax.experimental import pallas as pl
from jax.experimental.pallas import tpu as pltpu
from jax.experimental.pallas import tpu_sc as plsc

_N = 10000
_E = 320000
_D = 128
_T = 4
_DT = 32
_AVG_D_LOG = 3.4965075614664802
_EPS = 1e-5


_C = 2000
_U = 5
_NPAIR = _E // (2 * _C)
_GPC = _C // 16
_FNEG = -3.0e38
_FPOS = 3.0e38


def _retry_scatter(acc, idx, val, is_max):
    def cond(act):
        return plsc.all_reduce_population_count(act)[0] > 0

    def body(act):
        old = plsc.load_gather(acc, [idx])
        new = jnp.maximum(old, val) if is_max else jnp.minimum(old, val)
        plsc.store_scatter(acc, [idx], new, mask=act)
        chk = plsc.load_gather(acc, [idx])
        lost = (chk < val) if is_max else (chk > val)
        return act & lost

    lax.while_loop(cond, body, jnp.full((16,), True, jnp.bool_))


def _edge_phase(aT, src, dst):
    mesh = plsc.VectorSubcoreMesh(core_axis_name="c", subcore_axis_name="s")
    fvec = lambda v: jnp.full((16,), v, jnp.float32)

    @functools.partial(
        pl.kernel,
        out_type=(
            jax.ShapeDtypeStruct((4 * _D, _N), jnp.float32),
            jax.ShapeDtypeStruct((_N,), jnp.float32),
        ),
        mesh=mesh,
        scratch_types=(
            [pltpu.VMEM((_N,), jnp.float32) for _ in range(11)]
            + [pltpu.VMEM((_C,), jnp.int32) for _ in range(4)]
            + [pltpu.SemaphoreType.DMA for _ in range(4)]
        ),
        compiler_params=pltpu.CompilerParams(needs_layout_passes=False),
    )
    def body(aT_hbm, src_hbm, dst_hbm, agg_out, deg_out,
             sum0, sum1, sq0, sq1, mx0, mx1, mn0, mn1, arow0, arow1, degacc,
             srcA, dstA, srcB, dstB, semAs, semAd, semBs, semBd):
        wid = lax.axis_index("s") * 2 + lax.axis_index("c")

        def process(sbuf, dbuf, deg_on):
            def g_body(i, carry):
                base = i * (16 * _U)
                gs = []
                for u in range(_U):
                    sl = pl.ds(base + u * 16, 16)
                    s16 = sbuf[sl]
                    d16 = dbuf[sl]
                    _, lastm = plsc.scan_count(d16)
                    v0 = plsc.load_gather(arow0, [s16])
                    v1 = plsc.load_gather(arow1, [s16])
                    omx0 = plsc.load_gather(mx0, [d16])
                    omn0 = plsc.load_gather(mn0, [d16])
                    omx1 = plsc.load_gather(mx1, [d16])
                    omn1 = plsc.load_gather(mn1, [d16])
                    plsc.addupdate_scatter(sum0, [d16], v0)
                    plsc.addupdate_scatter(sq0, [d16], v0 * v0)
                    plsc.addupdate_scatter(sum1, [d16], v1)
                    plsc.addupdate_scatter(sq1, [d16], v1 * v1)
                    plsc.store_scatter(mx0, [d16], v0, mask=v0 > omx0)
                    plsc.store_scatter(mn0, [d16], v0, mask=v0 < omn0)
                    plsc.store_scatter(mx1, [d16], v1, mask=v1 > omx1)
                    plsc.store_scatter(mn1, [d16], v1, mask=v1 < omn1)
                    ndist = plsc.all_reduce_population_count(lastm)[0]
                    gs.append((d16, v0, v1, ndist))

                anydup = gs[0][3] < 16
                for u in range(1, _U):
                    anydup = anydup | (gs[u][3] < 16)

                @pl.when(anydup)
                def _():
                    for d16, v0, v1, _n in gs:
                        _retry_scatter(mx0, d16, v0, True)
                        _retry_scatter(mn0, d16, v0, False)
                        _retry_scatter(mx1, d16, v1, True)
                        _retry_scatter(mn1, d16, v1, False)

                return carry

            lax.fori_loop(0, _GPC // _U, g_body, 0)

            if deg_on:
                @pl.when(wid == 0)
                def _():
                    def deg_body(i, carry):
                        base = i * (16 * _U)
                        for u in range(_U):
                            d16 = dbuf[pl.ds(base + u * 16, 16)]
                            plsc.addupdate_scatter(degacc, [d16], fvec(1.0))
                        return carry

                    lax.fori_loop(0, _GPC // _U, deg_body, 0)

        for p in range(2):
            f0 = p * 64 + 2 * wid
            pltpu.sync_copy(aT_hbm.at[f0], arow0)
            pltpu.sync_copy(aT_hbm.at[f0 + 1], arow1)

            def init_body(i, carry):
                sl = pl.ds(i * 16, 16)
                z = fvec(0.0)
                sum0[sl] = z
                sum1[sl] = z
                sq0[sl] = z
                sq1[sl] = z
                mx0[sl] = fvec(_FNEG)
                mx1[sl] = fvec(_FNEG)
                mn0[sl] = fvec(_FPOS)
                mn1[sl] = fvec(_FPOS)
                if p == 0:
                    degacc[sl] = z
                return carry

            lax.fori_loop(0, _N // 16, init_body, 0)

            def dma(ch, buf_s, buf_d, sem_s, sem_d):
                esl = pl.ds(ch * _C, _C)
                a = pltpu.make_async_copy(src_hbm.at[esl], buf_s, sem_s)
                b = pltpu.make_async_copy(dst_hbm.at[esl], buf_d, sem_d)
                return a, b

            def start(ch, buf_s, buf_d, sem_s, sem_d):
                a, b = dma(ch, buf_s, buf_d, sem_s, sem_d)
                a.start()
                b.start()

            def wait(ch, buf_s, buf_d, sem_s, sem_d):
                a, b = dma(ch, buf_s, buf_d, sem_s, sem_d)
                a.wait()
                b.wait()

            start(0, srcA, dstA, semAs, semAd)

            def pair_body(i, carry):
                start(2 * i + 1, srcB, dstB, semBs, semBd)
                wait(2 * i, srcA, dstA, semAs, semAd)
                process(srcA, dstA, p == 0)

                @pl.when(i < _NPAIR - 1)
                def _():
                    start(2 * i + 2, srcA, dstA, semAs, semAd)

                wait(2 * i + 1, srcB, dstB, semBs, semBd)
                process(srcB, dstB, p == 0)
                return carry

            lax.fori_loop(0, _NPAIR, pair_body, 0)

            pltpu.sync_copy(sum0, agg_out.at[f0])
            pltpu.sync_copy(sum1, agg_out.at[f0 + 1])
            pltpu.sync_copy(sq0, agg_out.at[_D + f0])
            pltpu.sync_copy(sq1, agg_out.at[_D + f0 + 1])
            pltpu.sync_copy(mx0, agg_out.at[2 * _D + f0])
            pltpu.sync_copy(mx1, agg_out.at[2 * _D + f0 + 1])
            pltpu.sync_copy(mn0, agg_out.at[3 * _D + f0])
            pltpu.sync_copy(mn1, agg_out.at[3 * _D + f0 + 1])
            if p == 0:
                @pl.when(wid == 0)
                def _():
                    pltpu.sync_copy(degacc, deg_out)

    return body(aT, src, dst)


def _phase1_body(h_ref, wa_ref, ba_ref, wb_ref, a_ref, b_ref):
    h = h_ref[...]
    a_ref[...] = jnp.dot(h, wa_ref[...], preferred_element_type=jnp.float32) + ba_ref[...]
    b_ref[...] = jnp.dot(h, wb_ref[...], preferred_element_type=jnp.float32)


def _phase1(h, wa, ba, wb):
    return pl.pallas_call(
        _phase1_body,
        out_shape=(
            jax.ShapeDtypeStruct((_N, _D), jnp.float32),
            jax.ShapeDtypeStruct((_N, _D), jnp.float32),
        ),
    )(h, wa, ba, wb)


_NB = 5
_R = _N // _NB


def _tower_body(h_ref, b_ref, ssum_ref, ssq_ref, smax_ref, smin_ref,
                deg_ref, snorm_ref, w0_ref, w1_ref, w2_ref, w3_ref,
                bpost_ref, ppre_ref, sums_ref):
    nb = pl.program_id(0)
    deg = deg_ref[...]
    deg_c = jnp.maximum(deg, 1.0)
    has = deg > 0.0
    log_deg = jnp.log(deg_c + 1.0)
    amp_s = log_deg * (1.0 / _AVG_D_LOG)
    att_s = _AVG_D_LOG / log_deg
    snorm = snorm_ref[...]

    @pl.when(nb == 0)
    def _():
        sums_ref[...] = jnp.zeros_like(sums_ref)

    posts = []
    for t in range(_T):
        sl = slice(t * _DT, (t + 1) * _DT)
        bt = b_ref[:, sl]
        s1 = ssum_ref[:, sl]
        s2 = ssq_ref[:, sl]
        mean = (s1 + deg * bt) / deg_c
        sq = (s2 + 2.0 * bt * s1 + deg * bt * bt) / deg_c
        std = jnp.sqrt(jax.nn.relu(sq - mean * mean) + _EPS)
        mx = jnp.where(has, smax_ref[:, sl] + bt, 0.0)
        mn = jnp.where(has, smin_ref[:, sl] + bt, 0.0)
        agg = jnp.concatenate([mean, mx, mn, std], axis=1)
        post = jnp.dot(h_ref[:, sl], w0_ref[t], preferred_element_type=jnp.float32)
        post += jnp.dot(agg, w1_ref[t], preferred_element_type=jnp.float32)
        post += jnp.dot(agg * amp_s, w2_ref[t], preferred_element_type=jnp.float32)
        post += jnp.dot(agg * att_s, w3_ref[t], preferred_element_type=jnp.float32)
        posts.append((post + bpost_ref[:, sl]) * snorm)
    ppre = jnp.concatenate(posts, axis=1)
    ppre_ref[...] = ppre
    sums_ref[0:1, :] += jnp.sum(ppre, axis=0, keepdims=True)
    sums_ref[1:2, :] += jnp.sum(ppre * ppre, axis=0, keepdims=True)


def _phase2(h, b, ssum, ssq, smax, smin, deg, snorm, w0, w1, w2, w3, bpost):
    col = pl.BlockSpec((_R, _D), lambda i: (i, 0))
    vec = pl.BlockSpec((_R, 1), lambda i: (i, 0))
    fixed3 = pl.BlockSpec((_T, _DT, _DT), lambda i: (0, 0, 0))
    fixedw = pl.BlockSpec((_T, _D, _DT), lambda i: (0, 0, 0))
    return pl.pallas_call(
        _tower_body,
        grid=(_NB,),
        in_specs=[
            col, col, col, col, col, col, vec, vec,
            fixed3, fixedw, fixedw, fixedw,
            pl.BlockSpec((1, _D), lambda i: (0, 0)),
        ],
        out_specs=(
            col,
            pl.BlockSpec((8, _D), lambda i: (0, 0)),
        ),
        out_shape=(
            jax.ShapeDtypeStruct((_N, _D), jnp.float32),
            jax.ShapeDtypeStruct((8, _D), jnp.float32),
        ),
    )(h, b, ssum, ssq, smax, smin, deg, snorm, w0, w1, w2, w3, bpost)


def _phase3_body(h_ref, ppre_ref, sums_ref, gamma_ref, beta_ref, wmix_ref,
                 bmix_ref, out_ref):
    inv_n = 1.0 / _N
    mu = sums_ref[0:1, :] * inv_n
    var = sums_ref[1:2, :] * inv_n - mu * mu
    x = (ppre_ref[...] - mu) * lax.rsqrt(var + _EPS) * gamma_ref[...] \
        + beta_ref[...]
    h_out = jnp.dot(x, wmix_ref[...],
                    preferred_element_type=jnp.float32) + bmix_ref[...]
    h_out = jnp.where(h_out >= 0.0, h_out, 0.01 * h_out)
    out_ref[...] = h_ref[...] + h_out


def _phase3(h, ppre, sums, gamma, beta, wmix, bmix):
    return pl.pallas_call(
        _phase3_body,
        grid=(_NB,),
        in_specs=[
            pl.BlockSpec((_R, _D), lambda i: (i, 0)),
            pl.BlockSpec((_R, _D), lambda i: (i, 0)),
            pl.BlockSpec((8, _D), lambda i: (0, 0)),
            pl.BlockSpec((1, _D), lambda i: (0, 0)),
            pl.BlockSpec((1, _D), lambda i: (0, 0)),
            pl.BlockSpec((_D, _D), lambda i: (0, 0)),
            pl.BlockSpec((_D,), lambda i: (0,)),
        ],
        out_specs=pl.BlockSpec((_R, _D), lambda i: (i, 0)),
        out_shape=jax.ShapeDtypeStruct((_N, _D), jnp.float32),
    )(h, ppre, sums, gamma, beta, wmix, bmix)


def kernel(h, e, snorm_n, W_pre, b_pre, W_post, b_post, gamma, beta, W_mix,
           b_mix, edge_index):
    del e
    wa = jnp.zeros((_D, _D), jnp.float32)
    wb = jnp.zeros((_D, _D), jnp.float32)
    for t in range(_T):
        sl = slice(t * _DT, (t + 1) * _DT)
        wa = wa.at[sl, sl].set(W_pre[t, :_DT])
        wb = wb.at[sl, sl].set(W_pre[t, _DT:])
    ba = b_pre.reshape(_D)

    a, b = _phase1(h, wa, ba, wb)

    src = edge_index[0]
    dst = edge_index[1]
    aggT, deg = _edge_phase(a.T, src, dst)
    ssum = aggT[:_D].T
    ssq = aggT[_D:2 * _D].T
    smax = aggT[2 * _D:3 * _D].T
    smin = aggT[3 * _D:].T

    w0 = W_post[:, :_DT]
    w1 = W_post[:, _DT:_DT + _D]
    w2 = W_post[:, _DT + _D:_DT + 2 * _D]
    w3 = W_post[:, _DT + 2 * _D:]
    ppre, sums = _phase2(h, b, ssum, ssq, smax, smin, deg[:, None], snorm_n,
                         w0, w1, w2, w3, b_post.reshape(1, _D))
    return _phase3(h, ppre, sums, gamma.reshape(1, _D), beta.reshape(1, _D),
                   W_mix, b_mix)


# speedup vs baseline: 1.9737x; 1.2707x over previous
import functools

import jax
import jax.numpy as jnp
from jax import lax
from jax.experimental import pallas as pl
from jax.experimental.pallas import tpu as pltpu
from jax.experimental.pallas import tpu_sc as plsc

_N = 10000
_E = 320000
_D = 128
_T = 4
_DT = 32
_AVG_D_LOG = 3.4965075614664802
_EPS = 1e-5


_C = 2000
_U = 5
_NPAIR = _E // (2 * _C)
_GPC = _C // 16
_FNEG = -3.0e38
_FPOS = 3.0e38


def _retry_scatter(acc, idx, val, is_max):
    def cond(act):
        return plsc.all_reduce_population_count(act)[0] > 0

    def body(act):
        old = plsc.load_gather(acc, [idx])
        new = jnp.maximum(old, val) if is_max else jnp.minimum(old, val)
        plsc.store_scatter(acc, [idx], new, mask=act)
        chk = plsc.load_gather(acc, [idx])
        lost = (chk < val) if is_max else (chk > val)
        return act & lost

    lax.while_loop(cond, body, jnp.full((16,), True, jnp.bool_))


def _edge_phase(aT, src, dst):
    mesh = plsc.VectorSubcoreMesh(core_axis_name="c", subcore_axis_name="s")
    fvec = lambda v: jnp.full((16,), v, jnp.float32)

    @functools.partial(
        pl.kernel,
        out_type=(
            jax.ShapeDtypeStruct((4 * _D, _N), jnp.float32),
            jax.ShapeDtypeStruct((_N,), jnp.float32),
        ),
        mesh=mesh,
        scratch_types=(
            [pltpu.VMEM((_N,), jnp.float32) for _ in range(11)]
            + [pltpu.VMEM((_C,), jnp.int32) for _ in range(4)]
            + [pltpu.SemaphoreType.DMA for _ in range(4)]
        ),
        compiler_params=pltpu.CompilerParams(needs_layout_passes=False),
    )
    def body(aT_hbm, src_hbm, dst_hbm, agg_out, deg_out,
             sum0, sum1, sq0, sq1, mx0, mx1, mn0, mn1, arow0, arow1, degacc,
             srcA, dstA, srcB, dstB, semAs, semAd, semBs, semBd):
        wid = lax.axis_index("s") * 2 + lax.axis_index("c")

        def process(sbuf, dbuf, deg_on):
            def g_body(i, carry):
                base = i * (16 * _U)
                sds = []
                for u in range(_U):
                    sl = pl.ds(base + u * 16, 16)
                    sds.append((sbuf[sl], dbuf[sl]))
                gs = []
                for u in range(_U):
                    s16, d16 = sds[u]
                    _, lastm = plsc.scan_count(d16)
                    v0 = plsc.load_gather(arow0, [s16])
                    v1 = plsc.load_gather(arow1, [s16])
                    omx0 = plsc.load_gather(mx0, [d16])
                    omn0 = plsc.load_gather(mn0, [d16])
                    omx1 = plsc.load_gather(mx1, [d16])
                    omn1 = plsc.load_gather(mn1, [d16])
                    plsc.addupdate_scatter(sum0, [d16], v0)
                    plsc.addupdate_scatter(sq0, [d16], v0 * v0)
                    plsc.addupdate_scatter(sum1, [d16], v1)
                    plsc.addupdate_scatter(sq1, [d16], v1 * v1)
                    plsc.store_scatter(mx0, [d16], v0, mask=v0 > omx0)
                    plsc.store_scatter(mn0, [d16], v0, mask=v0 < omn0)
                    plsc.store_scatter(mx1, [d16], v1, mask=v1 > omx1)
                    plsc.store_scatter(mn1, [d16], v1, mask=v1 < omn1)
                    ndist = plsc.all_reduce_population_count(lastm)[0]
                    gs.append((d16, v0, v1, ndist))

                anydup = gs[0][3] < 16
                for u in range(1, _U):
                    anydup = anydup | (gs[u][3] < 16)

                @pl.when(anydup)
                def _():
                    for d16, v0, v1, _n in gs:
                        _retry_scatter(mx0, d16, v0, True)
                        _retry_scatter(mn0, d16, v0, False)
                        _retry_scatter(mx1, d16, v1, True)
                        _retry_scatter(mn1, d16, v1, False)

                return carry

            lax.fori_loop(0, _GPC // _U, g_body, 0)

            if deg_on:
                @pl.when(wid == 0)
                def _():
                    def deg_body(i, carry):
                        base = i * (16 * _U)
                        for u in range(_U):
                            d16 = dbuf[pl.ds(base + u * 16, 16)]
                            plsc.addupdate_scatter(degacc, [d16], fvec(1.0))
                        return carry

                    lax.fori_loop(0, _GPC // _U, deg_body, 0)

        for p in range(2):
            f0 = p * 64 + 2 * wid
            pltpu.sync_copy(aT_hbm.at[f0], arow0)
            pltpu.sync_copy(aT_hbm.at[f0 + 1], arow1)

            def init_body(i, carry):
                sl = pl.ds(i * 16, 16)
                z = fvec(0.0)
                sum0[sl] = z
                sum1[sl] = z
                sq0[sl] = z
                sq1[sl] = z
                mx0[sl] = fvec(_FNEG)
                mx1[sl] = fvec(_FNEG)
                mn0[sl] = fvec(_FPOS)
                mn1[sl] = fvec(_FPOS)
                if p == 0:
                    degacc[sl] = z
                return carry

            lax.fori_loop(0, _N // 16, init_body, 0)

            def dma(ch, buf_s, buf_d, sem_s, sem_d):
                esl = pl.ds(ch * _C, _C)
                a = pltpu.make_async_copy(src_hbm.at[esl], buf_s, sem_s)
                b = pltpu.make_async_copy(dst_hbm.at[esl], buf_d, sem_d)
                return a, b

            def start(ch, buf_s, buf_d, sem_s, sem_d):
                a, b = dma(ch, buf_s, buf_d, sem_s, sem_d)
                a.start()
                b.start()

            def wait(ch, buf_s, buf_d, sem_s, sem_d):
                a, b = dma(ch, buf_s, buf_d, sem_s, sem_d)
                a.wait()
                b.wait()

            start(0, srcA, dstA, semAs, semAd)

            def pair_body(i, carry):
                start(2 * i + 1, srcB, dstB, semBs, semBd)
                wait(2 * i, srcA, dstA, semAs, semAd)
                process(srcA, dstA, p == 0)

                @pl.when(i < _NPAIR - 1)
                def _():
                    start(2 * i + 2, srcA, dstA, semAs, semAd)

                wait(2 * i + 1, srcB, dstB, semBs, semBd)
                process(srcB, dstB, p == 0)
                return carry

            lax.fori_loop(0, _NPAIR, pair_body, 0)

            pltpu.sync_copy(sum0, agg_out.at[f0])
            pltpu.sync_copy(sum1, agg_out.at[f0 + 1])
            pltpu.sync_copy(sq0, agg_out.at[_D + f0])
            pltpu.sync_copy(sq1, agg_out.at[_D + f0 + 1])
            pltpu.sync_copy(mx0, agg_out.at[2 * _D + f0])
            pltpu.sync_copy(mx1, agg_out.at[2 * _D + f0 + 1])
            pltpu.sync_copy(mn0, agg_out.at[3 * _D + f0])
            pltpu.sync_copy(mn1, agg_out.at[3 * _D + f0 + 1])
            if p == 0:
                @pl.when(wid == 0)
                def _():
                    pltpu.sync_copy(degacc, deg_out)

    return body(aT, src, dst)


def _phase1_body(h_ref, wa_ref, ba_ref, wb_ref, a_ref, b_ref):
    h = h_ref[...]
    a_ref[...] = jnp.dot(h, wa_ref[...], preferred_element_type=jnp.float32) + ba_ref[...]
    b_ref[...] = jnp.dot(h, wb_ref[...], preferred_element_type=jnp.float32)


def _phase1(h, wa, ba, wb):
    return pl.pallas_call(
        _phase1_body,
        out_shape=(
            jax.ShapeDtypeStruct((_N, _D), jnp.float32),
            jax.ShapeDtypeStruct((_N, _D), jnp.float32),
        ),
    )(h, wa, ba, wb)


_NB = 5
_R = _N // _NB


def _tower_body(h_ref, b_ref, ssum_ref, ssq_ref, smax_ref, smin_ref,
                deg_ref, snorm_ref, w0_ref, w1_ref, w2_ref, w3_ref,
                bpost_ref, ppre_ref, sums_ref):
    nb = pl.program_id(0)
    deg = deg_ref[...]
    deg_c = jnp.maximum(deg, 1.0)
    has = deg > 0.0
    log_deg = jnp.log(deg_c + 1.0)
    amp_s = log_deg * (1.0 / _AVG_D_LOG)
    att_s = _AVG_D_LOG / log_deg
    snorm = snorm_ref[...]

    @pl.when(nb == 0)
    def _():
        sums_ref[...] = jnp.zeros_like(sums_ref)

    posts = []
    for t in range(_T):
        sl = slice(t * _DT, (t + 1) * _DT)
        bt = b_ref[:, sl]
        s1 = ssum_ref[:, sl]
        s2 = ssq_ref[:, sl]
        mean = (s1 + deg * bt) / deg_c
        sq = (s2 + 2.0 * bt * s1 + deg * bt * bt) / deg_c
        std = jnp.sqrt(jax.nn.relu(sq - mean * mean) + _EPS)
        mx = jnp.where(has, smax_ref[:, sl] + bt, 0.0)
        mn = jnp.where(has, smin_ref[:, sl] + bt, 0.0)
        agg = jnp.concatenate([mean, mx, mn, std], axis=1)
        post = jnp.dot(h_ref[:, sl], w0_ref[t], preferred_element_type=jnp.float32)
        post += jnp.dot(agg, w1_ref[t], preferred_element_type=jnp.float32)
        post += jnp.dot(agg * amp_s, w2_ref[t], preferred_element_type=jnp.float32)
        post += jnp.dot(agg * att_s, w3_ref[t], preferred_element_type=jnp.float32)
        posts.append((post + bpost_ref[:, sl]) * snorm)
    ppre = jnp.concatenate(posts, axis=1)
    ppre_ref[...] = ppre
    sums_ref[0:1, :] += jnp.sum(ppre, axis=0, keepdims=True)
    sums_ref[1:2, :] += jnp.sum(ppre * ppre, axis=0, keepdims=True)


def _phase2(h, b, ssum, ssq, smax, smin, deg, snorm, w0, w1, w2, w3, bpost):
    col = pl.BlockSpec((_R, _D), lambda i: (i, 0))
    vec = pl.BlockSpec((_R, 1), lambda i: (i, 0))
    fixed3 = pl.BlockSpec((_T, _DT, _DT), lambda i: (0, 0, 0))
    fixedw = pl.BlockSpec((_T, _D, _DT), lambda i: (0, 0, 0))
    return pl.pallas_call(
        _tower_body,
        grid=(_NB,),
        in_specs=[
            col, col, col, col, col, col, vec, vec,
            fixed3, fixedw, fixedw, fixedw,
            pl.BlockSpec((1, _D), lambda i: (0, 0)),
        ],
        out_specs=(
            col,
            pl.BlockSpec((8, _D), lambda i: (0, 0)),
        ),
        out_shape=(
            jax.ShapeDtypeStruct((_N, _D), jnp.float32),
            jax.ShapeDtypeStruct((8, _D), jnp.float32),
        ),
    )(h, b, ssum, ssq, smax, smin, deg, snorm, w0, w1, w2, w3, bpost)


def _phase3_body(h_ref, ppre_ref, sums_ref, gamma_ref, beta_ref, wmix_ref,
                 bmix_ref, out_ref):
    inv_n = 1.0 / _N
    mu = sums_ref[0:1, :] * inv_n
    var = sums_ref[1:2, :] * inv_n - mu * mu
    x = (ppre_ref[...] - mu) * lax.rsqrt(var + _EPS) * gamma_ref[...] \
        + beta_ref[...]
    h_out = jnp.dot(x, wmix_ref[...],
                    preferred_element_type=jnp.float32) + bmix_ref[...]
    h_out = jnp.where(h_out >= 0.0, h_out, 0.01 * h_out)
    out_ref[...] = h_ref[...] + h_out


def _phase3(h, ppre, sums, gamma, beta, wmix, bmix):
    return pl.pallas_call(
        _phase3_body,
        grid=(_NB,),
        in_specs=[
            pl.BlockSpec((_R, _D), lambda i: (i, 0)),
            pl.BlockSpec((_R, _D), lambda i: (i, 0)),
            pl.BlockSpec((8, _D), lambda i: (0, 0)),
            pl.BlockSpec((1, _D), lambda i: (0, 0)),
            pl.BlockSpec((1, _D), lambda i: (0, 0)),
            pl.BlockSpec((_D, _D), lambda i: (0, 0)),
            pl.BlockSpec((_D,), lambda i: (0,)),
        ],
        out_specs=pl.BlockSpec((_R, _D), lambda i: (i, 0)),
        out_shape=jax.ShapeDtypeStruct((_N, _D), jnp.float32),
    )(h, ppre, sums, gamma, beta, wmix, bmix)


def kernel(h, e, snorm_n, W_pre, b_pre, W_post, b_post, gamma, beta, W_mix,
           b_mix, edge_index):
    del e
    wa = jnp.zeros((_D, _D), jnp.float32)
    wb = jnp.zeros((_D, _D), jnp.float32)
    for t in range(_T):
        sl = slice(t * _DT, (t + 1) * _DT)
        wa = wa.at[sl, sl].set(W_pre[t, :_DT])
        wb = wb.at[sl, sl].set(W_pre[t, _DT:])
    ba = b_pre.reshape(_D)

    a, b = _phase1(h, wa, ba, wb)

    src = edge_index[0]
    dst = edge_index[1]
    aggT, deg = _edge_phase(a.T, src, dst)
    ssum = aggT[:_D].T
    ssq = aggT[_D:2 * _D].T
    smax = aggT[2 * _D:3 * _D].T
    smin = aggT[3 * _D:].T

    w0 = W_post[:, :_DT]
    w1 = W_post[:, _DT:_DT + _D]
    w2 = W_post[:, _DT + _D:_DT + 2 * _D]
    w3 = W_post[:, _DT + 2 * _D:]
    ppre, sums = _phase2(h, b, ssum, ssq, smax, smin, deg[:, None], snorm_n,
                         w0, w1, w2, w3, b_post.reshape(1, _D))
    return _phase3(h, ppre, sums, gamma.reshape(1, _D), beta.reshape(1, _D),
                   W_mix, b_mix)


# speedup vs baseline: 2.0073x; 1.0170x over previous
import functools

import jax
import jax.numpy as jnp
from jax import lax
from jax.experimental import pallas as pl
from jax.experimental.pallas import tpu as pltpu
from jax.experimental.pallas import tpu_sc as plsc

_N = 10000
_E = 320000
_D = 128
_T = 4
_DT = 32
_AVG_D_LOG = 3.4965075614664802
_EPS = 1e-5


_C = 2000
_U = 5
_NPAIR = _E // (2 * _C)
_GPC = _C // 16
_FNEG = -3.0e38
_FPOS = 3.0e38


def _retry_scatter(acc, idx, val, is_max):
    def cond(act):
        return plsc.all_reduce_population_count(act)[0] > 0

    def body(act):
        old = plsc.load_gather(acc, [idx])
        new = jnp.maximum(old, val) if is_max else jnp.minimum(old, val)
        plsc.store_scatter(acc, [idx], new, mask=act)
        chk = plsc.load_gather(acc, [idx])
        lost = (chk < val) if is_max else (chk > val)
        return act & lost

    lax.while_loop(cond, body, jnp.full((16,), True, jnp.bool_))


def _edge_phase(aT, src, dst):
    mesh = plsc.VectorSubcoreMesh(core_axis_name="c", subcore_axis_name="s")
    fvec = lambda v: jnp.full((16,), v, jnp.float32)

    @functools.partial(
        pl.kernel,
        out_type=(
            jax.ShapeDtypeStruct((4 * _D, _N), jnp.float32),
            jax.ShapeDtypeStruct((_N,), jnp.float32),
        ),
        mesh=mesh,
        scratch_types=(
            [pltpu.VMEM((_N,), jnp.float32) for _ in range(11)]
            + [pltpu.VMEM((_C,), jnp.int32) for _ in range(4)]
            + [pltpu.SemaphoreType.DMA for _ in range(4)]
        ),
        compiler_params=pltpu.CompilerParams(needs_layout_passes=False),
    )
    def body(aT_hbm, src_hbm, dst_hbm, agg_out, deg_out,
             sum0, sum1, sq0, sq1, mx0, mx1, mn0, mn1, arow0, arow1, degacc,
             srcA, dstA, srcB, dstB, semAs, semAd, semBs, semBd):
        wid = lax.axis_index("s") * 2 + lax.axis_index("c")

        def process(sbuf, dbuf, deg_on):
            def g_body(i, carry):
                base = i * (16 * _U)
                sds = []
                for u in range(_U):
                    sl = pl.ds(base + u * 16, 16)
                    sds.append((sbuf[sl], dbuf[sl]))
                vs = []
                for u in range(_U):
                    s16, _d = sds[u]
                    vs.append((plsc.load_gather(arow0, [s16]),
                               plsc.load_gather(arow1, [s16])))
                gs = []
                for u in range(_U):
                    _s, d16 = sds[u]
                    v0, v1 = vs[u]
                    _, lastm = plsc.scan_count(d16)
                    omx0 = plsc.load_gather(mx0, [d16])
                    omn0 = plsc.load_gather(mn0, [d16])
                    omx1 = plsc.load_gather(mx1, [d16])
                    omn1 = plsc.load_gather(mn1, [d16])
                    plsc.addupdate_scatter(sum0, [d16], v0)
                    plsc.addupdate_scatter(sq0, [d16], v0 * v0)
                    plsc.addupdate_scatter(sum1, [d16], v1)
                    plsc.addupdate_scatter(sq1, [d16], v1 * v1)
                    plsc.store_scatter(mx0, [d16], v0, mask=v0 > omx0)
                    plsc.store_scatter(mn0, [d16], v0, mask=v0 < omn0)
                    plsc.store_scatter(mx1, [d16], v1, mask=v1 > omx1)
                    plsc.store_scatter(mn1, [d16], v1, mask=v1 < omn1)
                    ndist = plsc.all_reduce_population_count(lastm)[0]
                    gs.append((d16, v0, v1, ndist))

                anydup = gs[0][3] < 16
                for u in range(1, _U):
                    anydup = anydup | (gs[u][3] < 16)

                @pl.when(anydup)
                def _():
                    for d16, v0, v1, _n in gs:
                        _retry_scatter(mx0, d16, v0, True)
                        _retry_scatter(mn0, d16, v0, False)
                        _retry_scatter(mx1, d16, v1, True)
                        _retry_scatter(mn1, d16, v1, False)

                return carry

            lax.fori_loop(0, _GPC // _U, g_body, 0)

            if deg_on:
                @pl.when(wid == 0)
                def _():
                    def deg_body(i, carry):
                        base = i * (16 * _U)
                        for u in range(_U):
                            d16 = dbuf[pl.ds(base + u * 16, 16)]
                            plsc.addupdate_scatter(degacc, [d16], fvec(1.0))
                        return carry

                    lax.fori_loop(0, _GPC // _U, deg_body, 0)

        for p in range(2):
            f0 = p * 64 + 2 * wid
            pltpu.sync_copy(aT_hbm.at[f0], arow0)
            pltpu.sync_copy(aT_hbm.at[f0 + 1], arow1)

            def init_body(i, carry):
                sl = pl.ds(i * 16, 16)
                z = fvec(0.0)
                sum0[sl] = z
                sum1[sl] = z
                sq0[sl] = z
                sq1[sl] = z
                mx0[sl] = fvec(_FNEG)
                mx1[sl] = fvec(_FNEG)
                mn0[sl] = fvec(_FPOS)
                mn1[sl] = fvec(_FPOS)
                if p == 0:
                    degacc[sl] = z
                return carry

            lax.fori_loop(0, _N // 16, init_body, 0)

            def dma(ch, buf_s, buf_d, sem_s, sem_d):
                esl = pl.ds(ch * _C, _C)
                a = pltpu.make_async_copy(src_hbm.at[esl], buf_s, sem_s)
                b = pltpu.make_async_copy(dst_hbm.at[esl], buf_d, sem_d)
                return a, b

            def start(ch, buf_s, buf_d, sem_s, sem_d):
                a, b = dma(ch, buf_s, buf_d, sem_s, sem_d)
                a.start()
                b.start()

            def wait(ch, buf_s, buf_d, sem_s, sem_d):
                a, b = dma(ch, buf_s, buf_d, sem_s, sem_d)
                a.wait()
                b.wait()

            start(0, srcA, dstA, semAs, semAd)

            def pair_body(i, carry):
                start(2 * i + 1, srcB, dstB, semBs, semBd)
                wait(2 * i, srcA, dstA, semAs, semAd)
                process(srcA, dstA, p == 0)

                @pl.when(i < _NPAIR - 1)
                def _():
                    start(2 * i + 2, srcA, dstA, semAs, semAd)

                wait(2 * i + 1, srcB, dstB, semBs, semBd)
                process(srcB, dstB, p == 0)
                return carry

            lax.fori_loop(0, _NPAIR, pair_body, 0)

            pltpu.sync_copy(sum0, agg_out.at[f0])
            pltpu.sync_copy(sum1, agg_out.at[f0 + 1])
            pltpu.sync_copy(sq0, agg_out.at[_D + f0])
            pltpu.sync_copy(sq1, agg_out.at[_D + f0 + 1])
            pltpu.sync_copy(mx0, agg_out.at[2 * _D + f0])
            pltpu.sync_copy(mx1, agg_out.at[2 * _D + f0 + 1])
            pltpu.sync_copy(mn0, agg_out.at[3 * _D + f0])
            pltpu.sync_copy(mn1, agg_out.at[3 * _D + f0 + 1])
            if p == 0:
                @pl.when(wid == 0)
                def _():
                    pltpu.sync_copy(degacc, deg_out)

    return body(aT, src, dst)


def _phase1_body(h_ref, wa_ref, ba_ref, wb_ref, a_ref, b_ref):
    h = h_ref[...]
    a_ref[...] = jnp.dot(h, wa_ref[...], preferred_element_type=jnp.float32) + ba_ref[...]
    b_ref[...] = jnp.dot(h, wb_ref[...], preferred_element_type=jnp.float32)


def _phase1(h, wa, ba, wb):
    return pl.pallas_call(
        _phase1_body,
        out_shape=(
            jax.ShapeDtypeStruct((_N, _D), jnp.float32),
            jax.ShapeDtypeStruct((_N, _D), jnp.float32),
        ),
    )(h, wa, ba, wb)


_NB = 5
_R = _N // _NB


def _tower_body(h_ref, b_ref, ssum_ref, ssq_ref, smax_ref, smin_ref,
                deg_ref, snorm_ref, w0_ref, w1_ref, w2_ref, w3_ref,
                bpost_ref, ppre_ref, sums_ref):
    nb = pl.program_id(0)
    deg = deg_ref[...]
    deg_c = jnp.maximum(deg, 1.0)
    has = deg > 0.0
    log_deg = jnp.log(deg_c + 1.0)
    amp_s = log_deg * (1.0 / _AVG_D_LOG)
    att_s = _AVG_D_LOG / log_deg
    snorm = snorm_ref[...]

    @pl.when(nb == 0)
    def _():
        sums_ref[...] = jnp.zeros_like(sums_ref)

    posts = []
    for t in range(_T):
        sl = slice(t * _DT, (t + 1) * _DT)
        bt = b_ref[:, sl]
        s1 = ssum_ref[:, sl]
        s2 = ssq_ref[:, sl]
        mean = (s1 + deg * bt) / deg_c
        sq = (s2 + 2.0 * bt * s1 + deg * bt * bt) / deg_c
        std = jnp.sqrt(jax.nn.relu(sq - mean * mean) + _EPS)
        mx = jnp.where(has, smax_ref[:, sl] + bt, 0.0)
        mn = jnp.where(has, smin_ref[:, sl] + bt, 0.0)
        agg = jnp.concatenate([mean, mx, mn, std], axis=1)
        post = jnp.dot(h_ref[:, sl], w0_ref[t], preferred_element_type=jnp.float32)
        post += jnp.dot(agg, w1_ref[t], preferred_element_type=jnp.float32)
        post += jnp.dot(agg * amp_s, w2_ref[t], preferred_element_type=jnp.float32)
        post += jnp.dot(agg * att_s, w3_ref[t], preferred_element_type=jnp.float32)
        posts.append((post + bpost_ref[:, sl]) * snorm)
    ppre = jnp.concatenate(posts, axis=1)
    ppre_ref[...] = ppre
    sums_ref[0:1, :] += jnp.sum(ppre, axis=0, keepdims=True)
    sums_ref[1:2, :] += jnp.sum(ppre * ppre, axis=0, keepdims=True)


def _phase2(h, b, ssum, ssq, smax, smin, deg, snorm, w0, w1, w2, w3, bpost):
    col = pl.BlockSpec((_R, _D), lambda i: (i, 0))
    vec = pl.BlockSpec((_R, 1), lambda i: (i, 0))
    fixed3 = pl.BlockSpec((_T, _DT, _DT), lambda i: (0, 0, 0))
    fixedw = pl.BlockSpec((_T, _D, _DT), lambda i: (0, 0, 0))
    return pl.pallas_call(
        _tower_body,
        grid=(_NB,),
        in_specs=[
            col, col, col, col, col, col, vec, vec,
            fixed3, fixedw, fixedw, fixedw,
            pl.BlockSpec((1, _D), lambda i: (0, 0)),
        ],
        out_specs=(
            col,
            pl.BlockSpec((8, _D), lambda i: (0, 0)),
        ),
        out_shape=(
            jax.ShapeDtypeStruct((_N, _D), jnp.float32),
            jax.ShapeDtypeStruct((8, _D), jnp.float32),
        ),
    )(h, b, ssum, ssq, smax, smin, deg, snorm, w0, w1, w2, w3, bpost)


def _phase3_body(h_ref, ppre_ref, sums_ref, gamma_ref, beta_ref, wmix_ref,
                 bmix_ref, out_ref):
    inv_n = 1.0 / _N
    mu = sums_ref[0:1, :] * inv_n
    var = sums_ref[1:2, :] * inv_n - mu * mu
    x = (ppre_ref[...] - mu) * lax.rsqrt(var + _EPS) * gamma_ref[...] \
        + beta_ref[...]
    h_out = jnp.dot(x, wmix_ref[...],
                    preferred_element_type=jnp.float32) + bmix_ref[...]
    h_out = jnp.where(h_out >= 0.0, h_out, 0.01 * h_out)
    out_ref[...] = h_ref[...] + h_out


def _phase3(h, ppre, sums, gamma, beta, wmix, bmix):
    return pl.pallas_call(
        _phase3_body,
        grid=(_NB,),
        in_specs=[
            pl.BlockSpec((_R, _D), lambda i: (i, 0)),
            pl.BlockSpec((_R, _D), lambda i: (i, 0)),
            pl.BlockSpec((8, _D), lambda i: (0, 0)),
            pl.BlockSpec((1, _D), lambda i: (0, 0)),
            pl.BlockSpec((1, _D), lambda i: (0, 0)),
            pl.BlockSpec((_D, _D), lambda i: (0, 0)),
            pl.BlockSpec((_D,), lambda i: (0,)),
        ],
        out_specs=pl.BlockSpec((_R, _D), lambda i: (i, 0)),
        out_shape=jax.ShapeDtypeStruct((_N, _D), jnp.float32),
    )(h, ppre, sums, gamma, beta, wmix, bmix)


def kernel(h, e, snorm_n, W_pre, b_pre, W_post, b_post, gamma, beta, W_mix,
           b_mix, edge_index):
    del e
    wa = jnp.zeros((_D, _D), jnp.float32)
    wb = jnp.zeros((_D, _D), jnp.float32)
    for t in range(_T):
        sl = slice(t * _DT, (t + 1) * _DT)
        wa = wa.at[sl, sl].set(W_pre[t, :_DT])
        wb = wb.at[sl, sl].set(W_pre[t, _DT:])
    ba = b_pre.reshape(_D)

    a, b = _phase1(h, wa, ba, wb)

    src = edge_index[0]
    dst = edge_index[1]
    aggT, deg = _edge_phase(a.T, src, dst)
    ssum = aggT[:_D].T
    ssq = aggT[_D:2 * _D].T
    smax = aggT[2 * _D:3 * _D].T
    smin = aggT[3 * _D:].T

    w0 = W_post[:, :_DT]
    w1 = W_post[:, _DT:_DT + _D]
    w2 = W_post[:, _DT + _D:_DT + 2 * _D]
    w3 = W_post[:, _DT + 2 * _D:]
    ppre, sums = _phase2(h, b, ssum, ssq, smax, smin, deg[:, None], snorm_n,
                         w0, w1, w2, w3, b_post.reshape(1, _D))
    return _phase3(h, ppre, sums, gamma.reshape(1, _D), beta.reshape(1, _D),
                   W_mix, b_mix)


# speedup vs baseline: 2.1459x; 1.0691x over previous
import functools

import jax
import jax.numpy as jnp
from jax import lax
from jax.experimental import pallas as pl
from jax.experimental.pallas import tpu as pltpu
from jax.experimental.pallas import tpu_sc as plsc

_N = 10000
_E = 320000
_D = 128
_T = 4
_DT = 32
_AVG_D_LOG = 3.4965075614664802
_EPS = 1e-5


_C = 2000
_U = 5
_NPAIR = _E // (2 * _C)
_GPC = _C // 16
_FNEG = -3.0e38
_FPOS = 3.0e38


def _retry_scatter(acc, idx, val, is_max):
    def cond(act):
        return plsc.all_reduce_population_count(act)[0] > 0

    def body(act):
        old = plsc.load_gather(acc, [idx])
        new = jnp.maximum(old, val) if is_max else jnp.minimum(old, val)
        plsc.store_scatter(acc, [idx], new, mask=act)
        chk = plsc.load_gather(acc, [idx])
        lost = (chk < val) if is_max else (chk > val)
        return act & lost

    lax.while_loop(cond, body, jnp.full((16,), True, jnp.bool_))


def _edge_phase(aT, src, dst):
    mesh = plsc.VectorSubcoreMesh(core_axis_name="c", subcore_axis_name="s")
    fvec = lambda v: jnp.full((16,), v, jnp.float32)

    @functools.partial(
        pl.kernel,
        out_type=(
            jax.ShapeDtypeStruct((4 * _D, _N), jnp.float32),
            jax.ShapeDtypeStruct((_N,), jnp.float32),
        ),
        mesh=mesh,
        scratch_types=(
            [pltpu.VMEM((_N,), jnp.float32) for _ in range(11)]
            + [pltpu.VMEM((_C,), jnp.int32) for _ in range(4)]
            + [pltpu.SemaphoreType.DMA for _ in range(4)]
        ),
        compiler_params=pltpu.CompilerParams(needs_layout_passes=False),
    )
    def body(aT_hbm, src_hbm, dst_hbm, agg_out, deg_out,
             sum0, sum1, sq0, sq1, mx0, mx1, mn0, mn1, arow0, arow1, degacc,
             srcA, dstA, srcB, dstB, semAs, semAd, semBs, semBd):
        wid = lax.axis_index("s") * 2 + lax.axis_index("c")

        def process(sbuf, dbuf, deg_on):
            def g_body(i, carry):
                base = i * (16 * _U)
                sds = []
                for u in range(_U):
                    sl = pl.ds(base + u * 16, 16)
                    sds.append((sbuf[sl], dbuf[sl]))
                vs = []
                for u in range(_U):
                    s16, _d = sds[u]
                    vs.append((plsc.load_gather(arow0, [s16]),
                               plsc.load_gather(arow1, [s16])))
                gs = []
                for u in range(_U):
                    _s, d16 = sds[u]
                    v0, v1 = vs[u]
                    _, lastm = plsc.scan_count(d16)
                    omx0 = plsc.load_gather(mx0, [d16])
                    omn0 = plsc.load_gather(mn0, [d16])
                    omx1 = plsc.load_gather(mx1, [d16])
                    omn1 = plsc.load_gather(mn1, [d16])
                    plsc.addupdate_scatter(sum0, [d16], v0)
                    plsc.addupdate_scatter(sq0, [d16], v0 * v0)
                    plsc.addupdate_scatter(sum1, [d16], v1)
                    plsc.addupdate_scatter(sq1, [d16], v1 * v1)
                    plsc.store_scatter(mx0, [d16], v0, mask=v0 > omx0)
                    plsc.store_scatter(mn0, [d16], v0, mask=v0 < omn0)
                    plsc.store_scatter(mx1, [d16], v1, mask=v1 > omx1)
                    plsc.store_scatter(mn1, [d16], v1, mask=v1 < omn1)
                    ndist = plsc.all_reduce_population_count(lastm)[0]
                    gs.append((d16, v0, v1, ndist))

                anydup = gs[0][3] < 16
                for u in range(1, _U):
                    anydup = anydup | (gs[u][3] < 16)

                @pl.when(anydup)
                def _():
                    for d16, v0, v1, _n in gs:
                        _retry_scatter(mx0, d16, v0, True)
                        _retry_scatter(mn0, d16, v0, False)
                        _retry_scatter(mx1, d16, v1, True)
                        _retry_scatter(mn1, d16, v1, False)

                return carry

            lax.fori_loop(0, _GPC // _U, g_body, 0)

            if deg_on:
                @pl.when(wid == 0)
                def _():
                    def deg_body(i, carry):
                        base = i * (16 * _U)
                        ds16 = [dbuf[pl.ds(base + u * 16, 16)]
                                for u in range(_U)]
                        for d16 in ds16:
                            plsc.addupdate_scatter(degacc, [d16], fvec(1.0))
                        return carry

                    lax.fori_loop(0, _GPC // _U, deg_body, 0)

        for p in range(2):
            f0 = p * 64 + 2 * wid
            pltpu.sync_copy(aT_hbm.at[f0], arow0)
            pltpu.sync_copy(aT_hbm.at[f0 + 1], arow1)

            def init_body(i, carry):
                sl = pl.ds(i * 16, 16)
                z = fvec(0.0)
                sum0[sl] = z
                sum1[sl] = z
                sq0[sl] = z
                sq1[sl] = z
                mx0[sl] = fvec(_FNEG)
                mx1[sl] = fvec(_FNEG)
                mn0[sl] = fvec(_FPOS)
                mn1[sl] = fvec(_FPOS)
                if p == 0:
                    degacc[sl] = z
                return carry

            lax.fori_loop(0, _N // 16, init_body, 0)

            def dma(ch, buf_s, buf_d, sem_s, sem_d):
                esl = pl.ds(ch * _C, _C)
                a = pltpu.make_async_copy(src_hbm.at[esl], buf_s, sem_s)
                b = pltpu.make_async_copy(dst_hbm.at[esl], buf_d, sem_d)
                return a, b

            def start(ch, buf_s, buf_d, sem_s, sem_d):
                a, b = dma(ch, buf_s, buf_d, sem_s, sem_d)
                a.start()
                b.start()

            def wait(ch, buf_s, buf_d, sem_s, sem_d):
                a, b = dma(ch, buf_s, buf_d, sem_s, sem_d)
                a.wait()
                b.wait()

            start(0, srcA, dstA, semAs, semAd)

            def pair_body(i, carry):
                start(2 * i + 1, srcB, dstB, semBs, semBd)
                wait(2 * i, srcA, dstA, semAs, semAd)
                process(srcA, dstA, p == 0)

                @pl.when(i < _NPAIR - 1)
                def _():
                    start(2 * i + 2, srcA, dstA, semAs, semAd)

                wait(2 * i + 1, srcB, dstB, semBs, semBd)
                process(srcB, dstB, p == 0)
                return carry

            lax.fori_loop(0, _NPAIR, pair_body, 0)

            pltpu.sync_copy(sum0, agg_out.at[f0])
            pltpu.sync_copy(sum1, agg_out.at[f0 + 1])
            pltpu.sync_copy(sq0, agg_out.at[_D + f0])
            pltpu.sync_copy(sq1, agg_out.at[_D + f0 + 1])
            pltpu.sync_copy(mx0, agg_out.at[2 * _D + f0])
            pltpu.sync_copy(mx1, agg_out.at[2 * _D + f0 + 1])
            pltpu.sync_copy(mn0, agg_out.at[3 * _D + f0])
            pltpu.sync_copy(mn1, agg_out.at[3 * _D + f0 + 1])
            if p == 0:
                @pl.when(wid == 0)
                def _():
                    pltpu.sync_copy(degacc, deg_out)

    return body(aT, src, dst)


def _phase1_body(h_ref, wa_ref, ba_ref, wb_ref, a_ref, b_ref):
    h = h_ref[...]
    a_ref[...] = jnp.dot(h, wa_ref[...], preferred_element_type=jnp.float32) + ba_ref[...]
    b_ref[...] = jnp.dot(h, wb_ref[...], preferred_element_type=jnp.float32)


def _phase1(h, wa, ba, wb):
    return pl.pallas_call(
        _phase1_body,
        grid=(_NB,),
        in_specs=[
            pl.BlockSpec((_R, _D), lambda i: (i, 0)),
            pl.BlockSpec((_D, _D), lambda i: (0, 0)),
            pl.BlockSpec((_D,), lambda i: (0,)),
            pl.BlockSpec((_D, _D), lambda i: (0, 0)),
        ],
        out_specs=(
            pl.BlockSpec((_R, _D), lambda i: (i, 0)),
            pl.BlockSpec((_R, _D), lambda i: (i, 0)),
        ),
        out_shape=(
            jax.ShapeDtypeStruct((_N, _D), jnp.float32),
            jax.ShapeDtypeStruct((_N, _D), jnp.float32),
        ),
    )(h, wa, ba, wb)


_NB = 5
_R = _N // _NB


def _tower_body(h_ref, b_ref, ssum_ref, ssq_ref, smax_ref, smin_ref,
                deg_ref, snorm_ref, w0_ref, w1_ref, w2_ref, w3_ref,
                bpost_ref, ppre_ref, sums_ref):
    nb = pl.program_id(0)
    deg = deg_ref[...]
    deg_c = jnp.maximum(deg, 1.0)
    has = deg > 0.0
    log_deg = jnp.log(deg_c + 1.0)
    amp_s = log_deg * (1.0 / _AVG_D_LOG)
    att_s = _AVG_D_LOG / log_deg
    snorm = snorm_ref[...]

    @pl.when(nb == 0)
    def _():
        sums_ref[...] = jnp.zeros_like(sums_ref)

    posts = []
    for t in range(_T):
        sl = slice(t * _DT, (t + 1) * _DT)
        bt = b_ref[:, sl]
        s1 = ssum_ref[:, sl]
        s2 = ssq_ref[:, sl]
        mean = (s1 + deg * bt) / deg_c
        sq = (s2 + 2.0 * bt * s1 + deg * bt * bt) / deg_c
        std = jnp.sqrt(jax.nn.relu(sq - mean * mean) + _EPS)
        mx = jnp.where(has, smax_ref[:, sl] + bt, 0.0)
        mn = jnp.where(has, smin_ref[:, sl] + bt, 0.0)
        agg = jnp.concatenate([mean, mx, mn, std], axis=1)
        post = jnp.dot(h_ref[:, sl], w0_ref[t], preferred_element_type=jnp.float32)
        post += jnp.dot(agg, w1_ref[t], preferred_element_type=jnp.float32)
        post += jnp.dot(agg * amp_s, w2_ref[t], preferred_element_type=jnp.float32)
        post += jnp.dot(agg * att_s, w3_ref[t], preferred_element_type=jnp.float32)
        posts.append((post + bpost_ref[:, sl]) * snorm)
    ppre = jnp.concatenate(posts, axis=1)
    ppre_ref[...] = ppre
    sums_ref[0:1, :] += jnp.sum(ppre, axis=0, keepdims=True)
    sums_ref[1:2, :] += jnp.sum(ppre * ppre, axis=0, keepdims=True)


def _phase2(h, b, ssum, ssq, smax, smin, deg, snorm, w0, w1, w2, w3, bpost):
    col = pl.BlockSpec((_R, _D), lambda i: (i, 0))
    vec = pl.BlockSpec((_R, 1), lambda i: (i, 0))
    fixed3 = pl.BlockSpec((_T, _DT, _DT), lambda i: (0, 0, 0))
    fixedw = pl.BlockSpec((_T, _D, _DT), lambda i: (0, 0, 0))
    return pl.pallas_call(
        _tower_body,
        grid=(_NB,),
        in_specs=[
            col, col, col, col, col, col, vec, vec,
            fixed3, fixedw, fixedw, fixedw,
            pl.BlockSpec((1, _D), lambda i: (0, 0)),
        ],
        out_specs=(
            col,
            pl.BlockSpec((8, _D), lambda i: (0, 0)),
        ),
        out_shape=(
            jax.ShapeDtypeStruct((_N, _D), jnp.float32),
            jax.ShapeDtypeStruct((8, _D), jnp.float32),
        ),
    )(h, b, ssum, ssq, smax, smin, deg, snorm, w0, w1, w2, w3, bpost)


def _phase3_body(h_ref, ppre_ref, sums_ref, gamma_ref, beta_ref, wmix_ref,
                 bmix_ref, out_ref):
    inv_n = 1.0 / _N
    mu = sums_ref[0:1, :] * inv_n
    var = sums_ref[1:2, :] * inv_n - mu * mu
    x = (ppre_ref[...] - mu) * lax.rsqrt(var + _EPS) * gamma_ref[...] \
        + beta_ref[...]
    h_out = jnp.dot(x, wmix_ref[...],
                    preferred_element_type=jnp.float32) + bmix_ref[...]
    h_out = jnp.where(h_out >= 0.0, h_out, 0.01 * h_out)
    out_ref[...] = h_ref[...] + h_out


def _phase3(h, ppre, sums, gamma, beta, wmix, bmix):
    return pl.pallas_call(
        _phase3_body,
        grid=(_NB,),
        in_specs=[
            pl.BlockSpec((_R, _D), lambda i: (i, 0)),
            pl.BlockSpec((_R, _D), lambda i: (i, 0)),
            pl.BlockSpec((8, _D), lambda i: (0, 0)),
            pl.BlockSpec((1, _D), lambda i: (0, 0)),
            pl.BlockSpec((1, _D), lambda i: (0, 0)),
            pl.BlockSpec((_D, _D), lambda i: (0, 0)),
            pl.BlockSpec((_D,), lambda i: (0,)),
        ],
        out_specs=pl.BlockSpec((_R, _D), lambda i: (i, 0)),
        out_shape=jax.ShapeDtypeStruct((_N, _D), jnp.float32),
    )(h, ppre, sums, gamma, beta, wmix, bmix)


def kernel(h, e, snorm_n, W_pre, b_pre, W_post, b_post, gamma, beta, W_mix,
           b_mix, edge_index):
    del e
    wa = jnp.zeros((_D, _D), jnp.float32)
    wb = jnp.zeros((_D, _D), jnp.float32)
    for t in range(_T):
        sl = slice(t * _DT, (t + 1) * _DT)
        wa = wa.at[sl, sl].set(W_pre[t, :_DT])
        wb = wb.at[sl, sl].set(W_pre[t, _DT:])
    ba = b_pre.reshape(_D)

    a, b = _phase1(h, wa, ba, wb)

    src = edge_index[0]
    dst = edge_index[1]
    aggT, deg = _edge_phase(a.T, src, dst)
    ssum = aggT[:_D].T
    ssq = aggT[_D:2 * _D].T
    smax = aggT[2 * _D:3 * _D].T
    smin = aggT[3 * _D:].T

    w0 = W_post[:, :_DT]
    w1 = W_post[:, _DT:_DT + _D]
    w2 = W_post[:, _DT + _D:_DT + 2 * _D]
    w3 = W_post[:, _DT + 2 * _D:]
    ppre, sums = _phase2(h, b, ssum, ssq, smax, smin, deg[:, None], snorm_n,
                         w0, w1, w2, w3, b_post.reshape(1, _D))
    return _phase3(h, ppre, sums, gamma.reshape(1, _D), beta.reshape(1, _D),
                   W_mix, b_mix)
